# Initial kernel scaffold; baseline (speedup 1.0000x reference)
import jax
import jax.numpy as jnp
from jax.experimental import pallas as pl


def kernel(raw_params):
    raise NotImplementedError("write your pallas kernel here")


# speedup vs baseline: 1.2121x; 1.2121x over previous
import jax
import jax.numpy as jnp
from jax import lax
from jax.experimental import pallas as pl
from jax.experimental.pallas import tpu as pltpu
from jax.experimental.pallas import tpu_sc as plsc

T = 8192
KC = 8
DC = 256
RPT = 513
NC, NS, L = 2, 16, 16
NW = NC * NS
TPW = T // NW
G = 16
NG = TPW // G

_SC_PARAMS = pltpu.CompilerParams(use_tc_tiling_on_sc=False,
                                  needs_layout_passes=False)


def _sc_body(v8_hbm, sel_hbm, comp_hbm,
             lidx_v, lg2d, prob_v, start_v, cidx_v, comp_v, sem):
    c = lax.axis_index("c")
    s = lax.axis_index("s")
    wid = s * NC + c
    wbase = wid * TPW
    iota = lax.iota(jnp.int32, L)

    def build_lidx(j, carry):
        lidx_v[pl.ds(j * L, L)] = (wbase + j * L + iota) * RPT
        return carry
    lax.fori_loop(0, TPW // L, build_lidx, 0)

    cp1 = pltpu.async_copy(v8_hbm.at[lidx_v.at[pl.ds(0, 128)]],
                           lg2d.at[pl.ds(0, 128)], sem)
    cp2 = pltpu.async_copy(v8_hbm.at[lidx_v.at[pl.ds(128, 128)]],
                           lg2d.at[pl.ds(128, 128)], sem)
    cp1.wait()
    cp2.wait()

    def softmax_g(g, carry):
        row = g * L + iota
        lk = [plsc.load_gather(lg2d, [row, jnp.full((L,), k, jnp.int32)])
              for k in range(KC)]
        m = lk[0]
        for k in range(1, KC):
            m = jnp.maximum(m, lk[k])
        ek = [jnp.exp(x - m) for x in lk]
        ssum = ek[0]
        for k in range(1, KC):
            ssum = ssum + ek[k]
        r = 1.0 / ssum
        pos0 = (g * L + iota) * KC
        for k in range(KC):
            plsc.store_scatter(prob_v, [pos0 + k], ek[k] * r)
        am = jnp.zeros((L,), jnp.int32)
        bm = lk[0]
        for k in range(1, KC):
            gt = lk[k] > bm
            am = jnp.where(gt, jnp.full((L,), k, jnp.int32), am)
            bm = jnp.maximum(bm, lk[k])
        start_v[pl.ds(g * L, L)] = (wbase + g * L + iota) * RPT + 1 + am * 64
        return carry
    lax.fori_loop(0, NG, softmax_g, 0)

    pltpu.sync_copy(prob_v, sel_hbm.at[pl.ds(wbase * KC, TPW * KC)])

    def gather_g(g, carry):
        for k in range(G):
            st = plsc.load_gather(start_v,
                                  [jnp.full((L,), g * G + k, jnp.int32)])
            for q in range(4):
                cidx_v[pl.ds(k * 64 + q * L, L)] = st + q * L + iota
        cps = [pltpu.async_copy(v8_hbm.at[cidx_v.at[pl.ds(k * 64, 64)]],
                                comp_v.at[pl.ds(k * 64, 64)], sem)
               for k in range(G)]
        for cp in cps:
            cp.wait()
        pltpu.sync_copy(comp_v,
                        comp_hbm.at[pl.ds((wbase + g * G) * 64, G * 64)])
        return carry
    lax.fori_loop(0, NG, gather_g, 0)


def _sc_route(v8):
    mesh = plsc.VectorSubcoreMesh(core_axis_name="c", subcore_axis_name="s",
                                  num_cores=NC, num_subcores=NS)
    return pl.kernel(
        _sc_body,
        out_type=(jax.ShapeDtypeStruct((T * KC,), jnp.float32),
                  jax.ShapeDtypeStruct((T * 64, 8), jnp.float32)),
        mesh=mesh,
        compiler_params=_SC_PARAMS,
        scratch_types=[
            pltpu.VMEM((TPW,), jnp.int32),
            pltpu.VMEM((TPW, 8), jnp.float32),
            pltpu.VMEM((TPW * KC,), jnp.float32),
            pltpu.VMEM((TPW,), jnp.int32),
            pltpu.VMEM((G * 64,), jnp.int32),
            pltpu.VMEM((G * 64, 8), jnp.float32),
            pltpu.SemaphoreType.DMA,
        ],
    )(v8)


def _tc_sample_body(comp_ref, eps_ref, out_ref):
    cb = comp_ref[...]
    mean = cb[:, :DC]
    logvar = cb[:, DC:]
    out_ref[...] = mean + jnp.exp(0.5 * logvar) * eps_ref[...]


def _tc_sample(comp, eps):
    bt = 256
    return pl.pallas_call(
        _tc_sample_body,
        grid=(T // bt,),
        in_specs=[pl.BlockSpec((bt, 2 * DC), lambda i: (i, 0)),
                  pl.BlockSpec((bt, DC), lambda i: (i, 0))],
        out_specs=pl.BlockSpec((bt, DC), lambda i: (i, 0)),
        out_shape=jax.ShapeDtypeStruct((T, DC), jnp.float32),
    )(comp, eps)


def kernel(raw_params):
    v8 = raw_params.reshape(T * RPT, 8)
    sel_flat, comp2d = _sc_route(v8)
    selector_params = sel_flat.reshape(T, KC)
    component_params = comp2d.reshape(T, 2 * DC)
    eps = jax.random.normal(jax.random.key(42), (T, DC), dtype=jnp.float32)
    samples = _tc_sample(component_params, eps)
    return (selector_params, component_params, samples)


# speedup vs baseline: 2.8107x; 2.3188x over previous
import jax
import jax.numpy as jnp
from jax import lax
from jax.experimental import pallas as pl
from jax.experimental.pallas import tpu as pltpu
from jax.experimental.pallas import tpu_sc as plsc

T = 8192
KC = 8
DC = 256
CW = KC + 2 * DC * KC
BT = 256
RC = 64


def _route_body(xt_ref, selT_ref, compT_ref):
    lg = [xt_ref[pl.ds(k, 1), :] for k in range(KC)]
    m = lg[0]
    for k in range(1, KC):
        m = jnp.maximum(m, lg[k])
    ek = [jnp.exp(x - m) for x in lg]
    ssum = ek[0]
    for k in range(1, KC):
        ssum = ssum + ek[k]
    r = 1.0 / ssum
    for k in range(KC):
        selT_ref[pl.ds(k, 1), :] = ek[k] * r
    am = jnp.zeros((1, BT), jnp.int32)
    bm = lg[0]
    for k in range(1, KC):
        gt = lg[k] > bm
        am = jnp.where(gt, k, am)
        bm = jnp.maximum(bm, lg[k])
    for rr in range(0, 2 * DC, RC):
        acc = jnp.where(am == 0, xt_ref[pl.ds(KC + rr, RC), :], 0.0)
        for a in range(1, KC):
            acc = jnp.where(am == a,
                            xt_ref[pl.ds(KC + 2 * DC * a + rr, RC), :], acc)
        compT_ref[pl.ds(rr, RC), :] = acc


def _route(xt):
    return pl.pallas_call(
        _route_body,
        grid=(T // BT,),
        in_specs=[pl.BlockSpec((CW, BT), lambda i: (0, i))],
        out_specs=[pl.BlockSpec((KC, BT), lambda i: (0, i)),
                   pl.BlockSpec((2 * DC, BT), lambda i: (0, i))],
        out_shape=[jax.ShapeDtypeStruct((KC, T), jnp.float32),
                   jax.ShapeDtypeStruct((2 * DC, T), jnp.float32)],
    )(xt)


def _sample_body(comp_ref, eps_ref, out_ref):
    cb = comp_ref[...]
    mean = cb[:, :DC]
    logvar = cb[:, DC:]
    out_ref[...] = mean + jnp.exp(0.5 * logvar) * eps_ref[...]


def _sample(comp, eps):
    bt = 256
    return pl.pallas_call(
        _sample_body,
        grid=(T // bt,),
        in_specs=[pl.BlockSpec((bt, 2 * DC), lambda i: (i, 0)),
                  pl.BlockSpec((bt, DC), lambda i: (i, 0))],
        out_specs=pl.BlockSpec((bt, DC), lambda i: (i, 0)),
        out_shape=jax.ShapeDtypeStruct((T, DC), jnp.float32),
    )(comp, eps)


def kernel(raw_params):
    xt = raw_params.T
    selT, compT = _route(xt)
    selector_params = selT.T
    component_params = compT.T
    eps = jax.random.normal(jax.random.key(42), (T, DC), dtype=jnp.float32)
    samples = _sample(component_params, eps)
    return (selector_params, component_params, samples)
